# baseline (speedup 1.0000x reference)
import functools

import jax
import jax.numpy as jnp
from jax import lax
from jax.experimental import pallas as pl
from jax.experimental.pallas import tpu as pltpu
from jax.experimental.pallas import tpu_sc as plsc

N = 100000
E = 1600000
G = 128
H = 32
HH = H // 2
C = 128
SK8 = 8
SK16 = 16
NSCH8 = 1564
NSCH16 = 782
EP = NSCH8 * SK8 * C
NPAD = EP - E
NTR = 16
NS = 16
NC = 2
ZR = 1024
NZF = N // ZR
NZT = N - NZF * ZR
NPT = N // NS
SPREAD = 128
PG = G * SPREAD
TB = 1000
TGRID = N // TB

_MESH = plsc.VectorSubcoreMesh(core_axis_name="c", subcore_axis_name="s")
_SC_PARAMS = pltpu.CompilerParams(use_tc_tiling_on_sc=False)


def _zero_fill(s, src_ref, dst_ref, extra):
    def zloop(i, carry):
        z = s + i * NS

        @pl.when(z < NZF)
        def _():
            pltpu.sync_copy(src_ref, dst_ref.at[pl.ds(z * ZR, ZR)])

        @pl.when(z == NZF)
        def _():
            pltpu.sync_copy(src_ref.at[pl.ds(0, NZT + extra)],
                            dst_ref.at[pl.ds(NZF * ZR, NZT + extra)])

        return carry

    lax.fori_loop(0, 7, zloop, 0)


@functools.partial(
    pl.kernel,
    out_type=(jax.ShapeDtypeStruct((N,), jnp.float32),
              jax.ShapeDtypeStruct((N,), jnp.float32)),
    mesh=_MESH,
    compiler_params=_SC_PARAMS,
    scratch_types=[
        pltpu.VMEM_SHARED((N + NTR,), jnp.float32),
        pltpu.VMEM((2, SK16, C), jnp.int32),
        pltpu.VMEM((C,), jnp.float32),
        pltpu.VMEM((ZR,), jnp.float32),
        pltpu.SemaphoreType.DMA,
        pltpu.SemaphoreType.DMA,
    ],
)
def _deg_kernel(ei_hbm, z1_hbm, out0_hbm, out1_hbm, acc, didx, ones, stg,
                sem, isem):
    c = lax.axis_index("c")
    s = lax.axis_index("s")
    for k in range(C // 16):
        ones[pl.ds(k * 16, 16)] = jnp.full((16,), 1.0, jnp.float32)
    pltpu.sync_copy(z1_hbm, stg)
    _zero_fill(s, stg, acc, NTR)
    plsc.subcore_barrier()

    base = c * (NSCH16 // NC) + s * 24 + jnp.minimum(s, 7)
    trip = 24 + jnp.where(s < 7, 1, 0)

    pltpu.async_copy(ei_hbm.at[1, base], didx.at[0], isem)

    def body(i, carry):
        b = lax.rem(i, 2)
        pltpu.make_async_copy(ei_hbm.at[1, base], didx.at[b], isem).wait()

        @pl.when(i + 1 < trip)
        def _():
            pltpu.async_copy(ei_hbm.at[1, base + i + 1], didx.at[1 - b], isem)

        hs = [pltpu.async_copy(ones, acc.at[didx.at[b, r]], sem, add=True)
              for r in range(SK16)]
        for h in hs:
            h.wait()
        return carry

    lax.fori_loop(0, trip, body, 0)
    plsc.subcore_barrier()

    def wloop(i, carry):
        z = s + i * NS

        @pl.when(z < NZF)
        def _():
            pltpu.sync_copy(acc.at[pl.ds(z * ZR, ZR)], stg)

            @pl.when(c == 0)
            def _():
                pltpu.sync_copy(stg, out0_hbm.at[pl.ds(z * ZR, ZR)])

            @pl.when(c == 1)
            def _():
                pltpu.sync_copy(stg, out1_hbm.at[pl.ds(z * ZR, ZR)])

        @pl.when(z == NZF)
        def _():
            pltpu.sync_copy(acc.at[pl.ds(NZF * ZR, NZT)], stg.at[pl.ds(0, NZT)])

            @pl.when(c == 0)
            def _():
                pltpu.sync_copy(stg.at[pl.ds(0, NZT)],
                                out0_hbm.at[pl.ds(NZF * ZR, NZT)])

            @pl.when(c == 1)
            def _():
                pltpu.sync_copy(stg.at[pl.ds(0, NZT)],
                                out1_hbm.at[pl.ds(NZF * ZR, NZT)])

        return carry

    lax.fori_loop(0, 7, wloop, 0)


_L1_SCRATCH = [
    pltpu.VMEM_SHARED((N + NTR, HH), jnp.float32),
    pltpu.VMEM((2, SK8, C), jnp.int32),
    pltpu.VMEM((2, SK8, C), jnp.int32),
    pltpu.VMEM((SK8, C, HH), jnp.float32),
    pltpu.SemaphoreType.DMA,
    pltpu.SemaphoreType.DMA,
    pltpu.SemaphoreType.DMA,
]


def _l1_body(xs_hbm, ei_hbm, z_hbm, out_hbm,
             acc, sidx, didx, rows, gsem, ssem, isem):
    c = lax.axis_index("c")
    s = lax.axis_index("s")

    def zloop(i, carry):
        z = s + i * NS

        @pl.when(z < NZF)
        def _():
            pltpu.sync_copy(z_hbm, acc.at[pl.ds(z * ZR, ZR), :])

        @pl.when(z == NZF)
        def _():
            pltpu.sync_copy(z_hbm.at[pl.ds(0, NZT + NTR), :],
                            acc.at[pl.ds(NZF * ZR, NZT + NTR), :])

        return carry

    lax.fori_loop(0, 7, zloop, 0)
    plsc.subcore_barrier()

    base = c * (NSCH8 // NC) + s * 48 + jnp.minimum(s, 14)
    trip = 48 + jnp.where(s < 14, 1, 0)

    pltpu.async_copy(ei_hbm.at[0, base], sidx.at[0], isem)
    pltpu.async_copy(ei_hbm.at[1, base], didx.at[0], isem)

    def body(i, carry):
        b = lax.rem(i, 2)
        pltpu.make_async_copy(ei_hbm.at[0, base], sidx.at[b], isem).wait()
        pltpu.make_async_copy(ei_hbm.at[1, base], didx.at[b], isem).wait()

        @pl.when(i + 1 < trip)
        def _():
            pltpu.async_copy(ei_hbm.at[0, base + i + 1], sidx.at[1 - b], isem)
            pltpu.async_copy(ei_hbm.at[1, base + i + 1], didx.at[1 - b], isem)

        HB = SK8 // 2
        g0 = [pltpu.async_copy(xs_hbm.at[sidx.at[b, r]], rows.at[r], gsem)
              for r in range(HB)]
        s0 = []
        for r in range(HB):
            g0[r].wait()
            s0.append(pltpu.async_copy(rows.at[r], acc.at[didx.at[b, r]],
                                       ssem, add=True))

        @pl.when(i > 0)
        def _():
            for r in range(HB, SK8):
                pltpu.make_async_copy(xs_hbm.at[sidx.at[b, r]], rows.at[r],
                                      ssem).wait()

        g1 = [pltpu.async_copy(xs_hbm.at[sidx.at[b, r]], rows.at[r], gsem)
              for r in range(HB, SK8)]
        for r in range(HB, SK8):
            g1[r - HB].wait()
            pltpu.async_copy(rows.at[r], acc.at[didx.at[b, r]], ssem, add=True)
        for h in s0:
            h.wait()
        return carry

    lax.fori_loop(0, trip, body, 0)
    for r in range(SK8 // 2, SK8):
        pltpu.make_async_copy(xs_hbm.at[sidx.at[0, r]], rows.at[r], ssem).wait()
    plsc.subcore_barrier()

    def wloop(i, carry):
        z = s + i * NS

        @pl.when(z < NZF)
        def _():
            pltpu.sync_copy(acc.at[pl.ds(z * ZR, ZR), :],
                            out_hbm.at[c, pl.ds(z * ZR, ZR), :])

        @pl.when(z == NZF)
        def _():
            pltpu.sync_copy(acc.at[pl.ds(NZF * ZR, NZT), :],
                            out_hbm.at[c, pl.ds(NZF * ZR, NZT), :])

        return carry

    lax.fori_loop(0, 7, wloop, 0)


_l1_kernel = functools.partial(
    pl.kernel,
    out_type=jax.ShapeDtypeStruct((NC, N, HH), jnp.float32),
    mesh=_MESH,
    compiler_params=_SC_PARAMS,
    scratch_types=_L1_SCRATCH,
)(_l1_body)


@functools.partial(
    pl.kernel,
    out_type=jax.ShapeDtypeStruct((NC, N, HH), jnp.float32),
    mesh=_MESH,
    compiler_params=_SC_PARAMS,
    scratch_types=[
        pltpu.VMEM_SHARED((N + NTR, HH), jnp.float32),
        pltpu.VMEM((2, SK8, C), jnp.int32),
        pltpu.VMEM((2, SK8, C), jnp.int32),
        pltpu.VMEM((SK8, C, HH), jnp.float32),
        pltpu.SemaphoreType.DMA,
        pltpu.SemaphoreType.DMA,
        pltpu.SemaphoreType.DMA,
    ],
)
def _layer_kernel(gs_hbm, ei_hbm, z_hbm, out_hbm,
                  acc, gidx, didx, rows, gsem, ssem, isem):
    c = lax.axis_index("c")
    s = lax.axis_index("s")
    coff = c * N

    def zloop(i, carry):
        z = s + i * NS

        @pl.when(z < NZF)
        def _():
            pltpu.sync_copy(z_hbm, acc.at[pl.ds(z * ZR, ZR), :])

        @pl.when(z == NZF)
        def _():
            pltpu.sync_copy(z_hbm.at[pl.ds(0, NZT + NTR), :],
                            acc.at[pl.ds(NZF * ZR, NZT + NTR), :])

        return carry

    lax.fori_loop(0, 7, zloop, 0)
    plsc.subcore_barrier()

    base = s * 97 + jnp.minimum(s, 12)
    trip = 97 + jnp.where(s < 12, 1, 0)

    pltpu.async_copy(ei_hbm.at[0, base], gidx.at[0], isem)
    pltpu.async_copy(ei_hbm.at[1, base], didx.at[0], isem)

    def body(i, carry):
        b = lax.rem(i, 2)
        pltpu.make_async_copy(ei_hbm.at[0, base], gidx.at[b], isem).wait()
        pltpu.make_async_copy(ei_hbm.at[1, base], didx.at[b], isem).wait()

        @pl.when(i + 1 < trip)
        def _():
            pltpu.async_copy(ei_hbm.at[0, base + i + 1], gidx.at[1 - b], isem)
            pltpu.async_copy(ei_hbm.at[1, base + i + 1], didx.at[1 - b], isem)

        for r in range(SK8):
            for k in range(C // 16):
                gidx[b, r, pl.ds(k * 16, 16)] = (
                    gidx[b, r, pl.ds(k * 16, 16)] + coff)
        HB = SK8 // 2
        g0 = [pltpu.async_copy(gs_hbm.at[gidx.at[b, r]], rows.at[r], gsem)
              for r in range(HB)]
        s0 = []
        for r in range(HB):
            g0[r].wait()
            s0.append(pltpu.async_copy(rows.at[r], acc.at[didx.at[b, r]], ssem,
                                       add=True))

        @pl.when(i > 0)
        def _():
            for r in range(HB, SK8):
                pltpu.make_async_copy(gs_hbm.at[gidx.at[b, r]], rows.at[r],
                                      ssem).wait()

        g1 = [pltpu.async_copy(gs_hbm.at[gidx.at[b, r]], rows.at[r], gsem)
              for r in range(HB, SK8)]
        for r in range(HB, SK8):
            g1[r - HB].wait()
            pltpu.async_copy(rows.at[r], acc.at[didx.at[b, r]], ssem, add=True)
        for h in s0:
            h.wait()
        return carry

    lax.fori_loop(0, trip, body, 0)
    for r in range(SK8 // 2, SK8):
        pltpu.make_async_copy(gs_hbm.at[gidx.at[0, r]], rows.at[r], ssem).wait()
    plsc.subcore_barrier()

    def wloop(i, carry):
        z = s + i * NS

        @pl.when(z < NZF)
        def _():
            pltpu.sync_copy(acc.at[pl.ds(z * ZR, ZR), :],
                            out_hbm.at[c, pl.ds(z * ZR, ZR), :])

        @pl.when(z == NZF)
        def _():
            pltpu.sync_copy(acc.at[pl.ds(NZF * ZR, NZT), :],
                            out_hbm.at[c, pl.ds(NZF * ZR, NZT), :])

        return carry

    lax.fori_loop(0, 7, wloop, 0)


@functools.partial(
    pl.kernel,
    out_type=jax.ShapeDtypeStruct((NC, PG, H), jnp.float32),
    mesh=_MESH,
    compiler_params=_SC_PARAMS,
    scratch_types=[
        pltpu.VMEM_SHARED((PG, H), jnp.float32),
        pltpu.VMEM((C, H), jnp.float32),
        pltpu.VMEM((C,), jnp.int32),
        pltpu.VMEM((1, C), jnp.int32),
        pltpu.SemaphoreType.DMA,
    ],
)
def _pool_kernel(agg_hbm, batch_hbm, z_hbm, out_hbm, pacc, nbuf, bbuf, tidx, sem):
    c = lax.axis_index("c")
    s = lax.axis_index("s")
    pltpu.sync_copy(z_hbm, pacc.at[pl.ds(s * ZR, ZR), :])
    plsc.subcore_barrier()

    half = N // NC
    nfull = half // C
    lb = s * 24 + jnp.minimum(s, 6)
    cnt = 24 + jnp.where(s < 6, 1, 0)

    def targets():
        for k in range(C // 16):
            b16 = bbuf[pl.ds(k * 16, 16)]
            tidx[0, pl.ds(k * 16, 16)] = (
                b16 * SPREAD + (k * 16 + lax.iota(jnp.int32, 16)))

    def body(i, carry):
        noff = c * half + (lb + i) * C
        pltpu.sync_copy(agg_hbm.at[pl.ds(noff, C), :], nbuf)
        pltpu.sync_copy(batch_hbm.at[pl.ds(noff, C)], bbuf)
        targets()
        pltpu.sync_copy(nbuf, pacc.at[tidx.at[0]], add=True)
        return carry

    lax.fori_loop(0, cnt, body, 0)

    @pl.when(s == NS - 1)
    def _():
        toff = c * half + nfull * C
        pltpu.sync_copy(z_hbm.at[pl.ds(0, C), :], nbuf)
        pltpu.sync_copy(agg_hbm.at[pl.ds(toff, 80), :], nbuf.at[pl.ds(0, 80), :])
        pltpu.sync_copy(batch_hbm.at[pl.ds(toff, 80)], bbuf.at[pl.ds(0, 80)])
        for k in range(5, 8):
            bbuf[pl.ds(k * 16, 16)] = jnp.zeros((16,), jnp.int32)
        targets()
        pltpu.sync_copy(nbuf, pacc.at[tidx.at[0]], add=True)

    plsc.subcore_barrier()
    pltpu.sync_copy(pacc.at[pl.ds(s * ZR, ZR), :],
                    out_hbm.at[c, pl.ds(s * ZR, ZR), :])


def _tc1(deg0, deg1, x, batch):
    def body(d0_ref, d1_ref, x_ref, b_ref, dinv_ref, xs_ref, cnt_ref, accs):
        i = pl.program_id(0)
        deg = d0_ref[...] + d1_ref[...] + 1.0
        dinv = lax.rsqrt(deg)
        dinv_ref[...] = dinv
        xs_ref[...] = jnp.concatenate(
            [dinv * x_ref[...], jnp.zeros((TB, HH - 4), jnp.float32)], axis=1)
        oh = (b_ref[...] ==
              lax.broadcasted_iota(jnp.int32, (TB, G), 1)).astype(jnp.float32)
        part = jnp.sum(oh, axis=0)[None, :]

        @pl.when(i == 0)
        def _():
            accs[...] = jnp.zeros_like(accs)

        accs[...] = accs[...] + part

        @pl.when(i == TGRID - 1)
        def _():
            cnt_ref[...] = accs[...]

    return pl.pallas_call(
        body,
        grid=(TGRID,),
        in_specs=[pl.BlockSpec((TB, 1), lambda i: (i, 0)),
                  pl.BlockSpec((TB, 1), lambda i: (i, 0)),
                  pl.BlockSpec((TB, 4), lambda i: (i, 0)),
                  pl.BlockSpec((TB, 1), lambda i: (i, 0))],
        out_specs=[pl.BlockSpec((TB, 1), lambda i: (i, 0)),
                   pl.BlockSpec((TB, HH), lambda i: (i, 0)),
                   pl.BlockSpec((1, G), lambda i: (0, 0))],
        out_shape=[jax.ShapeDtypeStruct((N, 1), jnp.float32),
                   jax.ShapeDtypeStruct((N, HH), jnp.float32),
                   jax.ShapeDtypeStruct((1, G), jnp.float32)],
        scratch_shapes=[pltpu.VMEM((1, G), jnp.float32)],
    )(deg0, deg1, x, batch)


def _tc2a(accx, xs, dinv, W1, b1, W2):
    def body(a_ref, x_ref, d_ref, w1_ref, b1_ref, w2_ref, o_ref):
        d = d_ref[...]
        t = (a_ref[0] + a_ref[1] + x_ref[...])[:, :4] * d
        h = jnp.maximum(
            jnp.dot(t, w1_ref[...], preferred_element_type=jnp.float32)
            + b1_ref[...], 0.0)
        gs = d * jnp.dot(h, w2_ref[...], preferred_element_type=jnp.float32)
        o_ref[0] = gs[:, :HH]
        o_ref[1] = gs[:, HH:]

    return pl.pallas_call(
        body,
        grid=(TGRID,),
        in_specs=[pl.BlockSpec((2, TB, HH), lambda i: (0, i, 0)),
                  pl.BlockSpec((TB, HH), lambda i: (i, 0)),
                  pl.BlockSpec((TB, 1), lambda i: (i, 0)),
                  pl.BlockSpec((4, H), lambda i: (0, 0)),
                  pl.BlockSpec((H,), lambda i: (0,)),
                  pl.BlockSpec((H, H), lambda i: (0, 0))],
        out_specs=pl.BlockSpec((2, TB, HH), lambda i: (0, i, 0)),
        out_shape=jax.ShapeDtypeStruct((2, N, HH), jnp.float32),
    )(accx, xs, dinv, W1, b1, W2)


def _tc_mid(accl, gsp, dinv, Wn, bprev):
    def body(a_ref, g_ref, d_ref, w_ref, b_ref, o_ref):
        d = d_ref[...]
        al = a_ref[0] + g_ref[0]
        ah = a_ref[1] + g_ref[1]
        a = jnp.concatenate([al, ah], axis=1) * d
        h = jnp.maximum(a + b_ref[...], 0.0)
        g = jnp.dot(h, w_ref[...], preferred_element_type=jnp.float32)
        gs = d * g
        o_ref[0] = gs[:, :HH]
        o_ref[1] = gs[:, HH:]

    return pl.pallas_call(
        body,
        grid=(TGRID,),
        in_specs=[pl.BlockSpec((2, TB, HH), lambda i: (0, i, 0)),
                  pl.BlockSpec((2, TB, HH), lambda i: (0, i, 0)),
                  pl.BlockSpec((TB, 1), lambda i: (i, 0)),
                  pl.BlockSpec((H, H), lambda i: (0, 0)),
                  pl.BlockSpec((H,), lambda i: (0,))],
        out_specs=pl.BlockSpec((2, TB, HH), lambda i: (0, i, 0)),
        out_shape=jax.ShapeDtypeStruct((2, N, HH), jnp.float32),
    )(accl, gsp, dinv, Wn, bprev)


def _tc4(accl, gsp, dinv):
    def body(a_ref, g_ref, d_ref, o_ref):
        d = d_ref[...]
        al = a_ref[0] + g_ref[0]
        ah = a_ref[1] + g_ref[1]
        o_ref[...] = jnp.concatenate([al, ah], axis=1) * d

    return pl.pallas_call(
        body,
        grid=(TGRID,),
        in_specs=[pl.BlockSpec((2, TB, HH), lambda i: (0, i, 0)),
                  pl.BlockSpec((2, TB, HH), lambda i: (0, i, 0)),
                  pl.BlockSpec((TB, 1), lambda i: (i, 0))],
        out_specs=pl.BlockSpec((TB, H), lambda i: (i, 0)),
        out_shape=jax.ShapeDtypeStruct((N, H), jnp.float32),
    )(accl, gsp, dinv)


def _tc5(pool4, cnt_col, b3, Wp, bp, Wl, bl):
    def body(p_ref, c_ref, b3_ref, wp_ref, bp_ref, wl_ref, bl_ref, o_ref):
        P = p_ref[...]
        ps = jnp.sum(P, axis=2)
        ps = ps[0] + ps[1]
        cv = c_ref[...]
        pooled = (ps + cv * b3_ref[...]) / jnp.maximum(cv, 1.0)
        p2 = jnp.dot(pooled, wp_ref[...],
                     preferred_element_type=jnp.float32) + bp_ref[...]
        o_ref[...] = jnp.dot(p2, wl_ref[...],
                             preferred_element_type=jnp.float32) + bl_ref[...]

    return pl.pallas_call(
        body,
        out_shape=jax.ShapeDtypeStruct((G, 4), jnp.float32),
    )(pool4, cnt_col, b3, Wp, bp, Wl, bl)


def kernel(x, W1, b1, W2, b2, W3, b3, Wp, bp, Wl, bl, edge_index, batch):
    pad_src = jnp.zeros((NPAD,), jnp.int32)
    pad_dst = N + (jnp.arange(NPAD, dtype=jnp.int32) % NTR)
    eip = jnp.concatenate([edge_index, jnp.stack([pad_src, pad_dst])], axis=1)
    ei8 = eip.reshape(2, NSCH8, SK8, C)
    ei16 = eip.reshape(2, NSCH16, SK16, C)
    z16 = jnp.zeros((ZR, HH), jnp.float32)
    z1 = jnp.zeros((ZR,), jnp.float32)
    z32 = jnp.zeros((ZR, H), jnp.float32)

    deg0, deg1 = _deg_kernel(ei16, z1)
    dinv, xs, cnt = _tc1(deg0.reshape(N, 1), deg1.reshape(N, 1), x,
                         batch.reshape(N, 1))
    accx = _l1_kernel(xs, ei8, z16)
    gs1 = _tc2a(accx, xs, dinv, W1, b1, W2)
    acc2 = _layer_kernel(gs1.reshape(NC * N, HH), ei8, z16)
    gs2 = _tc_mid(acc2, gs1, dinv, W3, b2)
    acc3 = _layer_kernel(gs2.reshape(NC * N, HH), ei8, z16)
    agg3 = _tc4(acc3, gs2, dinv)
    poolp = _pool_kernel(agg3, batch, z32)
    out = _tc5(poolp.reshape(NC, G, SPREAD, H), cnt.reshape(G, 1),
               b3, Wp, bp, Wl, bl)
    return out


# speedup vs baseline: 1.0603x; 1.0603x over previous
import functools

import jax
import jax.numpy as jnp
from jax import lax
from jax.experimental import pallas as pl
from jax.experimental.pallas import tpu as pltpu
from jax.experimental.pallas import tpu_sc as plsc

N = 100000
E = 1600000
G = 128
H = 32
HH = H // 2
C = 128
SK8 = 8
SK16 = 16
NSCH8 = 1564
NSCH16 = 782
EP = NSCH8 * SK8 * C
NPAD = EP - E
NTR = 16
NS = 16
NC = 2
ZR = 1024
NZF = N // ZR
NZT = N - NZF * ZR
NPT = N // NS
SPREAD = 128
PG = G * SPREAD
TB = 1000
TGRID = N // TB

_MESH = plsc.VectorSubcoreMesh(core_axis_name="c", subcore_axis_name="s")
_SC_PARAMS = pltpu.CompilerParams(use_tc_tiling_on_sc=False)


def _zero_fill(s, src_ref, dst_ref, extra):
    def zloop(i, carry):
        z = s + i * NS

        @pl.when(z < NZF)
        def _():
            pltpu.sync_copy(src_ref, dst_ref.at[pl.ds(z * ZR, ZR)])

        @pl.when(z == NZF)
        def _():
            pltpu.sync_copy(src_ref.at[pl.ds(0, NZT + extra)],
                            dst_ref.at[pl.ds(NZF * ZR, NZT + extra)])

        return carry

    lax.fori_loop(0, 7, zloop, 0)


@functools.partial(
    pl.kernel,
    out_type=(jax.ShapeDtypeStruct((N,), jnp.float32),
              jax.ShapeDtypeStruct((N,), jnp.float32)),
    mesh=_MESH,
    compiler_params=_SC_PARAMS,
    scratch_types=[
        pltpu.VMEM_SHARED((N + NTR,), jnp.float32),
        pltpu.VMEM((2, SK16, C), jnp.int32),
        pltpu.VMEM((C,), jnp.float32),
        pltpu.VMEM((ZR,), jnp.float32),
        pltpu.SemaphoreType.DMA,
        pltpu.SemaphoreType.DMA,
    ],
)
def _deg_kernel(ei_hbm, z1_hbm, out0_hbm, out1_hbm, acc, didx, ones, stg,
                sem, isem):
    c = lax.axis_index("c")
    s = lax.axis_index("s")
    for k in range(C // 16):
        ones[pl.ds(k * 16, 16)] = jnp.full((16,), 1.0, jnp.float32)
    pltpu.sync_copy(z1_hbm, stg)
    _zero_fill(s, stg, acc, NTR)
    plsc.subcore_barrier()

    base = c * (NSCH16 // NC) + s * 24 + jnp.minimum(s, 7)
    trip = 24 + jnp.where(s < 7, 1, 0)

    pltpu.async_copy(ei_hbm.at[1, base], didx.at[0], isem)

    def body(i, carry):
        b = lax.rem(i, 2)
        pltpu.make_async_copy(ei_hbm.at[1, base], didx.at[b], isem).wait()

        @pl.when(i + 1 < trip)
        def _():
            pltpu.async_copy(ei_hbm.at[1, base + i + 1], didx.at[1 - b], isem)

        hs = [pltpu.async_copy(ones, acc.at[didx.at[b, r]], sem, add=True)
              for r in range(SK16)]
        for h in hs:
            h.wait()
        return carry

    lax.fori_loop(0, trip, body, 0)
    plsc.subcore_barrier()

    def wloop(i, carry):
        z = s + i * NS

        @pl.when(z < NZF)
        def _():
            pltpu.sync_copy(acc.at[pl.ds(z * ZR, ZR)], stg)

            @pl.when(c == 0)
            def _():
                pltpu.sync_copy(stg, out0_hbm.at[pl.ds(z * ZR, ZR)])

            @pl.when(c == 1)
            def _():
                pltpu.sync_copy(stg, out1_hbm.at[pl.ds(z * ZR, ZR)])

        @pl.when(z == NZF)
        def _():
            pltpu.sync_copy(acc.at[pl.ds(NZF * ZR, NZT)], stg.at[pl.ds(0, NZT)])

            @pl.when(c == 0)
            def _():
                pltpu.sync_copy(stg.at[pl.ds(0, NZT)],
                                out0_hbm.at[pl.ds(NZF * ZR, NZT)])

            @pl.when(c == 1)
            def _():
                pltpu.sync_copy(stg.at[pl.ds(0, NZT)],
                                out1_hbm.at[pl.ds(NZF * ZR, NZT)])

        return carry

    lax.fori_loop(0, 7, wloop, 0)


_L1_SCRATCH = [
    pltpu.VMEM_SHARED((N + NTR, HH), jnp.float32),
    pltpu.VMEM((2, SK8, C), jnp.int32),
    pltpu.VMEM((2, SK8, C), jnp.int32),
    pltpu.VMEM((SK8, C, HH), jnp.float32),
    pltpu.SemaphoreType.DMA,
    pltpu.SemaphoreType.DMA,
    pltpu.SemaphoreType.DMA,
]


def _l1_body(xs_hbm, ei_hbm, z_hbm, out_hbm,
             acc, sidx, didx, rows, gsem, ssem, isem):
    c = lax.axis_index("c")
    s = lax.axis_index("s")

    def zloop(i, carry):
        z = s + i * NS

        @pl.when(z < NZF)
        def _():
            pltpu.sync_copy(z_hbm, acc.at[pl.ds(z * ZR, ZR), :])

        @pl.when(z == NZF)
        def _():
            pltpu.sync_copy(z_hbm.at[pl.ds(0, NZT + NTR), :],
                            acc.at[pl.ds(NZF * ZR, NZT + NTR), :])

        return carry

    lax.fori_loop(0, 7, zloop, 0)
    plsc.subcore_barrier()

    base = c * (NSCH8 // NC) + s * 48 + jnp.minimum(s, 14)
    trip = 48 + jnp.where(s < 14, 1, 0)

    pltpu.async_copy(ei_hbm.at[0, base], sidx.at[0], isem)
    pltpu.async_copy(ei_hbm.at[1, base], didx.at[0], isem)

    def body(i, carry):
        b = lax.rem(i, 2)
        pltpu.make_async_copy(ei_hbm.at[0, base], sidx.at[b], isem).wait()
        pltpu.make_async_copy(ei_hbm.at[1, base], didx.at[b], isem).wait()

        @pl.when(i + 1 < trip)
        def _():
            pltpu.async_copy(ei_hbm.at[0, base + i + 1], sidx.at[1 - b], isem)
            pltpu.async_copy(ei_hbm.at[1, base + i + 1], didx.at[1 - b], isem)

        gh = [pltpu.async_copy(xs_hbm.at[sidx.at[b, r]], rows.at[r], gsem)
              for r in range(SK8)]
        sh = []
        for r in range(SK8):
            gh[r].wait()
            sh.append(pltpu.async_copy(rows.at[r], acc.at[didx.at[b, r]],
                                       ssem, add=True))
        for h in sh:
            h.wait()
        return carry

    lax.fori_loop(0, trip, body, 0)
    plsc.subcore_barrier()

    def wloop(i, carry):
        z = s + i * NS

        @pl.when(z < NZF)
        def _():
            pltpu.sync_copy(acc.at[pl.ds(z * ZR, ZR), :],
                            out_hbm.at[c, pl.ds(z * ZR, ZR), :])

        @pl.when(z == NZF)
        def _():
            pltpu.sync_copy(acc.at[pl.ds(NZF * ZR, NZT), :],
                            out_hbm.at[c, pl.ds(NZF * ZR, NZT), :])

        return carry

    lax.fori_loop(0, 7, wloop, 0)


_l1_kernel = functools.partial(
    pl.kernel,
    out_type=jax.ShapeDtypeStruct((NC, N, HH), jnp.float32),
    mesh=_MESH,
    compiler_params=_SC_PARAMS,
    scratch_types=_L1_SCRATCH,
)(_l1_body)


@functools.partial(
    pl.kernel,
    out_type=jax.ShapeDtypeStruct((NC, N, HH), jnp.float32),
    mesh=_MESH,
    compiler_params=_SC_PARAMS,
    scratch_types=[
        pltpu.VMEM_SHARED((N + NTR, HH), jnp.float32),
        pltpu.VMEM((2, SK8, C), jnp.int32),
        pltpu.VMEM((2, SK8, C), jnp.int32),
        pltpu.VMEM((SK8, C, HH), jnp.float32),
        pltpu.SemaphoreType.DMA,
        pltpu.SemaphoreType.DMA,
        pltpu.SemaphoreType.DMA,
    ],
)
def _layer_kernel(gs_hbm, ei_hbm, z_hbm, out_hbm,
                  acc, gidx, didx, rows, gsem, ssem, isem):
    c = lax.axis_index("c")
    s = lax.axis_index("s")
    coff = c * N

    def zloop(i, carry):
        z = s + i * NS

        @pl.when(z < NZF)
        def _():
            pltpu.sync_copy(z_hbm, acc.at[pl.ds(z * ZR, ZR), :])

        @pl.when(z == NZF)
        def _():
            pltpu.sync_copy(z_hbm.at[pl.ds(0, NZT + NTR), :],
                            acc.at[pl.ds(NZF * ZR, NZT + NTR), :])

        return carry

    lax.fori_loop(0, 7, zloop, 0)
    plsc.subcore_barrier()

    base = s * 97 + jnp.minimum(s, 12)
    trip = 97 + jnp.where(s < 12, 1, 0)

    pltpu.async_copy(ei_hbm.at[0, base], gidx.at[0], isem)
    pltpu.async_copy(ei_hbm.at[1, base], didx.at[0], isem)

    def body(i, carry):
        b = lax.rem(i, 2)
        pltpu.make_async_copy(ei_hbm.at[0, base], gidx.at[b], isem).wait()
        pltpu.make_async_copy(ei_hbm.at[1, base], didx.at[b], isem).wait()

        @pl.when(i + 1 < trip)
        def _():
            pltpu.async_copy(ei_hbm.at[0, base + i + 1], gidx.at[1 - b], isem)
            pltpu.async_copy(ei_hbm.at[1, base + i + 1], didx.at[1 - b], isem)

        for r in range(SK8):
            for k in range(C // 16):
                gidx[b, r, pl.ds(k * 16, 16)] = (
                    gidx[b, r, pl.ds(k * 16, 16)] + coff)
        gh = [pltpu.async_copy(gs_hbm.at[gidx.at[b, r]], rows.at[r], gsem)
              for r in range(SK8)]
        sh = []
        for r in range(SK8):
            gh[r].wait()
            sh.append(pltpu.async_copy(rows.at[r], acc.at[didx.at[b, r]], ssem,
                                       add=True))
        for h in sh:
            h.wait()
        return carry

    lax.fori_loop(0, trip, body, 0)
    plsc.subcore_barrier()

    def wloop(i, carry):
        z = s + i * NS

        @pl.when(z < NZF)
        def _():
            pltpu.sync_copy(acc.at[pl.ds(z * ZR, ZR), :],
                            out_hbm.at[c, pl.ds(z * ZR, ZR), :])

        @pl.when(z == NZF)
        def _():
            pltpu.sync_copy(acc.at[pl.ds(NZF * ZR, NZT), :],
                            out_hbm.at[c, pl.ds(NZF * ZR, NZT), :])

        return carry

    lax.fori_loop(0, 7, wloop, 0)


@functools.partial(
    pl.kernel,
    out_type=jax.ShapeDtypeStruct((NC, PG, H), jnp.float32),
    mesh=_MESH,
    compiler_params=_SC_PARAMS,
    scratch_types=[
        pltpu.VMEM_SHARED((PG, H), jnp.float32),
        pltpu.VMEM((C, H), jnp.float32),
        pltpu.VMEM((C,), jnp.int32),
        pltpu.VMEM((1, C), jnp.int32),
        pltpu.SemaphoreType.DMA,
    ],
)
def _pool_kernel(agg_hbm, batch_hbm, z_hbm, out_hbm, pacc, nbuf, bbuf, tidx, sem):
    c = lax.axis_index("c")
    s = lax.axis_index("s")
    pltpu.sync_copy(z_hbm, pacc.at[pl.ds(s * ZR, ZR), :])
    plsc.subcore_barrier()

    half = N // NC
    nfull = half // C
    lb = s * 24 + jnp.minimum(s, 6)
    cnt = 24 + jnp.where(s < 6, 1, 0)

    def targets():
        for k in range(C // 16):
            b16 = bbuf[pl.ds(k * 16, 16)]
            tidx[0, pl.ds(k * 16, 16)] = (
                b16 * SPREAD + (k * 16 + lax.iota(jnp.int32, 16)))

    def body(i, carry):
        noff = c * half + (lb + i) * C
        pltpu.sync_copy(agg_hbm.at[pl.ds(noff, C), :], nbuf)
        pltpu.sync_copy(batch_hbm.at[pl.ds(noff, C)], bbuf)
        targets()
        pltpu.sync_copy(nbuf, pacc.at[tidx.at[0]], add=True)
        return carry

    lax.fori_loop(0, cnt, body, 0)

    @pl.when(s == NS - 1)
    def _():
        toff = c * half + nfull * C
        pltpu.sync_copy(z_hbm.at[pl.ds(0, C), :], nbuf)
        pltpu.sync_copy(agg_hbm.at[pl.ds(toff, 80), :], nbuf.at[pl.ds(0, 80), :])
        pltpu.sync_copy(batch_hbm.at[pl.ds(toff, 80)], bbuf.at[pl.ds(0, 80)])
        for k in range(5, 8):
            bbuf[pl.ds(k * 16, 16)] = jnp.zeros((16,), jnp.int32)
        targets()
        pltpu.sync_copy(nbuf, pacc.at[tidx.at[0]], add=True)

    plsc.subcore_barrier()
    pltpu.sync_copy(pacc.at[pl.ds(s * ZR, ZR), :],
                    out_hbm.at[c, pl.ds(s * ZR, ZR), :])


def _tc1(deg0, deg1, x, batch):
    def body(d0_ref, d1_ref, x_ref, b_ref, dinv_ref, xs_ref, cnt_ref, accs):
        i = pl.program_id(0)
        deg = d0_ref[...] + d1_ref[...] + 1.0
        dinv = lax.rsqrt(deg)
        dinv_ref[...] = dinv
        xs_ref[...] = jnp.concatenate(
            [dinv * x_ref[...], jnp.zeros((TB, HH - 4), jnp.float32)], axis=1)
        oh = (b_ref[...] ==
              lax.broadcasted_iota(jnp.int32, (TB, G), 1)).astype(jnp.float32)
        part = jnp.sum(oh, axis=0)[None, :]

        @pl.when(i == 0)
        def _():
            accs[...] = jnp.zeros_like(accs)

        accs[...] = accs[...] + part

        @pl.when(i == TGRID - 1)
        def _():
            cnt_ref[...] = accs[...]

    return pl.pallas_call(
        body,
        grid=(TGRID,),
        in_specs=[pl.BlockSpec((TB, 1), lambda i: (i, 0)),
                  pl.BlockSpec((TB, 1), lambda i: (i, 0)),
                  pl.BlockSpec((TB, 4), lambda i: (i, 0)),
                  pl.BlockSpec((TB, 1), lambda i: (i, 0))],
        out_specs=[pl.BlockSpec((TB, 1), lambda i: (i, 0)),
                   pl.BlockSpec((TB, HH), lambda i: (i, 0)),
                   pl.BlockSpec((1, G), lambda i: (0, 0))],
        out_shape=[jax.ShapeDtypeStruct((N, 1), jnp.float32),
                   jax.ShapeDtypeStruct((N, HH), jnp.float32),
                   jax.ShapeDtypeStruct((1, G), jnp.float32)],
        scratch_shapes=[pltpu.VMEM((1, G), jnp.float32)],
    )(deg0, deg1, x, batch)


def _tc2a(accx, xs, dinv, W1, b1, W2):
    def body(a_ref, x_ref, d_ref, w1_ref, b1_ref, w2_ref, o_ref):
        d = d_ref[...]
        t = (a_ref[0] + a_ref[1] + x_ref[...])[:, :4] * d
        h = jnp.maximum(
            jnp.dot(t, w1_ref[...], preferred_element_type=jnp.float32)
            + b1_ref[...], 0.0)
        gs = d * jnp.dot(h, w2_ref[...], preferred_element_type=jnp.float32)
        o_ref[0] = gs[:, :HH]
        o_ref[1] = gs[:, HH:]

    return pl.pallas_call(
        body,
        grid=(TGRID,),
        in_specs=[pl.BlockSpec((2, TB, HH), lambda i: (0, i, 0)),
                  pl.BlockSpec((TB, HH), lambda i: (i, 0)),
                  pl.BlockSpec((TB, 1), lambda i: (i, 0)),
                  pl.BlockSpec((4, H), lambda i: (0, 0)),
                  pl.BlockSpec((H,), lambda i: (0,)),
                  pl.BlockSpec((H, H), lambda i: (0, 0))],
        out_specs=pl.BlockSpec((2, TB, HH), lambda i: (0, i, 0)),
        out_shape=jax.ShapeDtypeStruct((2, N, HH), jnp.float32),
    )(accx, xs, dinv, W1, b1, W2)


def _tc_mid(accl, gsp, dinv, Wn, bprev):
    def body(a_ref, g_ref, d_ref, w_ref, b_ref, o_ref):
        d = d_ref[...]
        al = a_ref[0] + g_ref[0]
        ah = a_ref[1] + g_ref[1]
        a = jnp.concatenate([al, ah], axis=1) * d
        h = jnp.maximum(a + b_ref[...], 0.0)
        g = jnp.dot(h, w_ref[...], preferred_element_type=jnp.float32)
        gs = d * g
        o_ref[0] = gs[:, :HH]
        o_ref[1] = gs[:, HH:]

    return pl.pallas_call(
        body,
        grid=(TGRID,),
        in_specs=[pl.BlockSpec((2, TB, HH), lambda i: (0, i, 0)),
                  pl.BlockSpec((2, TB, HH), lambda i: (0, i, 0)),
                  pl.BlockSpec((TB, 1), lambda i: (i, 0)),
                  pl.BlockSpec((H, H), lambda i: (0, 0)),
                  pl.BlockSpec((H,), lambda i: (0,))],
        out_specs=pl.BlockSpec((2, TB, HH), lambda i: (0, i, 0)),
        out_shape=jax.ShapeDtypeStruct((2, N, HH), jnp.float32),
    )(accl, gsp, dinv, Wn, bprev)


def _tc4(accl, gsp, dinv):
    def body(a_ref, g_ref, d_ref, o_ref):
        d = d_ref[...]
        al = a_ref[0] + g_ref[0]
        ah = a_ref[1] + g_ref[1]
        o_ref[...] = jnp.concatenate([al, ah], axis=1) * d

    return pl.pallas_call(
        body,
        grid=(TGRID,),
        in_specs=[pl.BlockSpec((2, TB, HH), lambda i: (0, i, 0)),
                  pl.BlockSpec((2, TB, HH), lambda i: (0, i, 0)),
                  pl.BlockSpec((TB, 1), lambda i: (i, 0))],
        out_specs=pl.BlockSpec((TB, H), lambda i: (i, 0)),
        out_shape=jax.ShapeDtypeStruct((N, H), jnp.float32),
    )(accl, gsp, dinv)


def _tc5(pool4, cnt_col, b3, Wp, bp, Wl, bl):
    def body(p_ref, c_ref, b3_ref, wp_ref, bp_ref, wl_ref, bl_ref, o_ref):
        P = p_ref[...]
        ps = jnp.sum(P, axis=2)
        ps = ps[0] + ps[1]
        cv = c_ref[...]
        pooled = (ps + cv * b3_ref[...]) / jnp.maximum(cv, 1.0)
        p2 = jnp.dot(pooled, wp_ref[...],
                     preferred_element_type=jnp.float32) + bp_ref[...]
        o_ref[...] = jnp.dot(p2, wl_ref[...],
                             preferred_element_type=jnp.float32) + bl_ref[...]

    return pl.pallas_call(
        body,
        out_shape=jax.ShapeDtypeStruct((G, 4), jnp.float32),
    )(pool4, cnt_col, b3, Wp, bp, Wl, bl)


def kernel(x, W1, b1, W2, b2, W3, b3, Wp, bp, Wl, bl, edge_index, batch):
    pad_src = jnp.zeros((NPAD,), jnp.int32)
    pad_dst = N + (jnp.arange(NPAD, dtype=jnp.int32) % NTR)
    eip = jnp.concatenate([edge_index, jnp.stack([pad_src, pad_dst])], axis=1)
    ei8 = eip.reshape(2, NSCH8, SK8, C)
    ei16 = eip.reshape(2, NSCH16, SK16, C)
    z16 = jnp.zeros((ZR, HH), jnp.float32)
    z1 = jnp.zeros((ZR,), jnp.float32)
    z32 = jnp.zeros((ZR, H), jnp.float32)

    deg0, deg1 = _deg_kernel(ei16, z1)
    dinv, xs, cnt = _tc1(deg0.reshape(N, 1), deg1.reshape(N, 1), x,
                         batch.reshape(N, 1))
    accx = _l1_kernel(xs, ei8, z16)
    gs1 = _tc2a(accx, xs, dinv, W1, b1, W2)
    acc2 = _layer_kernel(gs1.reshape(NC * N, HH), ei8, z16)
    gs2 = _tc_mid(acc2, gs1, dinv, W3, b2)
    acc3 = _layer_kernel(gs2.reshape(NC * N, HH), ei8, z16)
    agg3 = _tc4(acc3, gs2, dinv)
    poolp = _pool_kernel(agg3, batch, z32)
    out = _tc5(poolp.reshape(NC, G, SPREAD, H), cnt.reshape(G, 1),
               b3, Wp, bp, Wl, bl)
    return out
